# baseline (speedup 1.0000x reference)
import jax
import jax.numpy as jnp
from jax import lax
from jax.experimental import pallas as pl
from jax.experimental.pallas import tpu as pltpu
from jax.experimental.pallas import tpu_sc as plsc
import functools

VOCAB = 1000000
EMB = 64
B = 16384
L = 50
LP = 56

NC = 2
NS = 16
NW = NC * NS

BPW = B // NW
NBUF = 8


def _make_gather():
    mesh = plsc.VectorSubcoreMesh(
        core_axis_name="c", subcore_axis_name="s",
        num_cores=NC, num_subcores=NS)

    @functools.partial(
        pl.kernel,
        out_type=jax.ShapeDtypeStruct((B, LP, 2 * EMB), jnp.float32),
        mesh=mesh,
        scratch_types=[
            pltpu.VMEM((BPW, LP), jnp.int32),
            pltpu.VMEM((NBUF, LP, EMB), jnp.float32),
            pltpu.SemaphoreType.DMA((NBUF,)),
            pltpu.SemaphoreType.DMA((NBUF,)),
        ],
        compiler_params=pltpu.CompilerParams(use_tc_tiling_on_sc=False),
    )
    def gather_kernel(idx_hbm, table_hbm, out_hbm, idx_v, rows_v, gsem, osem):
        cid = lax.axis_index("c")
        sid = lax.axis_index("s")
        wid = sid * NC + cid
        b0 = wid * BPW
        pltpu.sync_copy(idx_hbm.at[pl.ds(b0, BPW)], idx_v)

        def start_gather(r, bb):
            pltpu.make_async_copy(
                table_hbm.at[idx_v.at[bb]], rows_v.at[r], gsem.at[r]).start()

        def wait_gather(r, bb):
            pltpu.make_async_copy(
                table_hbm.at[idx_v.at[bb]], rows_v.at[r], gsem.at[r]).wait()

        def start_out(r, bb):
            pltpu.make_async_copy(
                rows_v.at[r],
                out_hbm.at[b0 + bb, :, pl.ds(0, EMB)], osem.at[r]).start()

        def wait_out(r, bb):
            pltpu.make_async_copy(
                rows_v.at[r],
                out_hbm.at[b0 + bb, :, pl.ds(0, EMB)], osem.at[r]).wait()

        for r in range(NBUF):
            start_gather(r, r)

        @pl.loop(0, BPW - NBUF, step=NBUF)
        def _(j0):
            for r in range(NBUF):
                wait_gather(r, j0 + r)
                start_out(r, j0 + r)
            for r in range(NBUF):
                wait_out(r, j0 + r)
                start_gather(r, j0 + NBUF + r)

        for r in range(NBUF):
            wait_gather(r, BPW - NBUF + r)
            start_out(r, BPW - NBUF + r)
        for r in range(NBUF):
            wait_out(r, BPW - NBUF + r)

    return gather_kernel


_gather = _make_gather()


def kernel(text_vec, w2v_table):
    idx = jnp.pad(text_vec.astype(jnp.int32), ((0, 0), (0, LP - L)))
    out = _gather(idx, w2v_table)
    return out[:, :L, :EMB]


# speedup vs baseline: 1.0000x; 1.0000x over previous
import jax
import jax.numpy as jnp
from jax import lax
from jax.experimental import pallas as pl
from jax.experimental.pallas import tpu as pltpu
from jax.experimental.pallas import tpu_sc as plsc
import functools

VOCAB = 1000000
EMB = 64
B = 16384
L = 50
LP = 56

NC = 2
NS = 16
NW = NC * NS

G = 2 * LP
NGRP = B * LP // G
GPW = NGRP // NW
NBUF = 8


def _make_gather():
    mesh = plsc.VectorSubcoreMesh(
        core_axis_name="c", subcore_axis_name="s",
        num_cores=NC, num_subcores=NS)

    @functools.partial(
        pl.kernel,
        out_type=jax.ShapeDtypeStruct((NGRP, G, 2 * EMB), jnp.float32),
        mesh=mesh,
        scratch_types=[
            pltpu.VMEM((GPW, G), jnp.int32),
            pltpu.VMEM((NBUF, G, EMB), jnp.float32),
            pltpu.SemaphoreType.DMA((NBUF,)),
            pltpu.SemaphoreType.DMA((NBUF,)),
        ],
        compiler_params=pltpu.CompilerParams(use_tc_tiling_on_sc=False),
    )
    def gather_kernel(idx_hbm, table_hbm, out_hbm, idx_v, rows_v, gsem, osem):
        cid = lax.axis_index("c")
        sid = lax.axis_index("s")
        wid = sid * NC + cid
        g0 = wid * GPW
        pltpu.sync_copy(idx_hbm.at[pl.ds(g0, GPW)], idx_v)

        def start_gather(r, g):
            pltpu.make_async_copy(
                table_hbm.at[idx_v.at[g]], rows_v.at[r], gsem.at[r]).start()

        def wait_gather(r, g):
            pltpu.make_async_copy(
                table_hbm.at[idx_v.at[g]], rows_v.at[r], gsem.at[r]).wait()

        def start_out(r, g):
            pltpu.make_async_copy(
                rows_v.at[r],
                out_hbm.at[g0 + g, :, pl.ds(0, EMB)], osem.at[r]).start()

        def wait_out(r, g):
            pltpu.make_async_copy(
                rows_v.at[r],
                out_hbm.at[g0 + g, :, pl.ds(0, EMB)], osem.at[r]).wait()

        for r in range(NBUF):
            start_gather(r, r)

        @pl.loop(0, GPW - NBUF, step=NBUF)
        def _(j0):
            for r in range(NBUF):
                wait_gather(r, j0 + r)
                start_out(r, j0 + r)
            for r in range(NBUF):
                wait_out(r, j0 + r)
                start_gather(r, j0 + NBUF + r)

        for r in range(NBUF):
            wait_gather(r, GPW - NBUF + r)
            start_out(r, GPW - NBUF + r)
        for r in range(NBUF):
            wait_out(r, GPW - NBUF + r)

    return gather_kernel


_gather = _make_gather()


def kernel(text_vec, w2v_table):
    idx = jnp.pad(text_vec.astype(jnp.int32),
                  ((0, 0), (0, LP - L))).reshape(NGRP, G)
    out = _gather(idx, w2v_table)
    return out.reshape(B, LP, 2 * EMB)[:, :L, :EMB]


# speedup vs baseline: 1.0005x; 1.0005x over previous
import jax
import jax.numpy as jnp
from jax import lax
from jax.experimental import pallas as pl
from jax.experimental.pallas import tpu as pltpu
from jax.experimental.pallas import tpu_sc as plsc
import functools

VOCAB = 1000000
EMB = 64
B = 16384
L = 50
LP = 56

NC = 2
NS = 16
NW = NC * NS

G = 128
NGRP = B * LP // G
GPW = NGRP // NW
NBUF = 8


def _make_gather():
    mesh = plsc.VectorSubcoreMesh(
        core_axis_name="c", subcore_axis_name="s",
        num_cores=NC, num_subcores=NS)

    @functools.partial(
        pl.kernel,
        out_type=jax.ShapeDtypeStruct((NGRP, G, 2 * EMB), jnp.float32),
        mesh=mesh,
        scratch_types=[
            pltpu.VMEM((GPW, G), jnp.int32),
            pltpu.VMEM((NBUF, G, EMB), jnp.float32),
            pltpu.SemaphoreType.DMA((NBUF,)),
            pltpu.SemaphoreType.DMA((NBUF,)),
        ],
        compiler_params=pltpu.CompilerParams(use_tc_tiling_on_sc=False),
    )
    def gather_kernel(idx_hbm, table_hbm, out_hbm, idx_v, rows_v, gsem, osem):
        cid = lax.axis_index("c")
        sid = lax.axis_index("s")
        wid = sid * NC + cid
        g0 = wid * GPW
        pltpu.sync_copy(idx_hbm.at[pl.ds(g0, GPW)], idx_v)

        def start_gather(r, g):
            pltpu.make_async_copy(
                table_hbm.at[idx_v.at[g]], rows_v.at[r], gsem.at[r]).start()

        def wait_gather(r, g):
            pltpu.make_async_copy(
                table_hbm.at[idx_v.at[g]], rows_v.at[r], gsem.at[r]).wait()

        def start_out(r, g):
            pltpu.make_async_copy(
                rows_v.at[r],
                out_hbm.at[g0 + g, :, pl.ds(0, EMB)], osem.at[r]).start()

        def wait_out(r, g):
            pltpu.make_async_copy(
                rows_v.at[r],
                out_hbm.at[g0 + g, :, pl.ds(0, EMB)], osem.at[r]).wait()

        for r in range(NBUF):
            start_gather(r, r)

        @pl.loop(0, GPW - NBUF, step=NBUF)
        def _(j0):
            for r in range(NBUF):
                wait_gather(r, j0 + r)
                start_out(r, j0 + r)
            for r in range(NBUF):
                wait_out(r, j0 + r)
                start_gather(r, j0 + NBUF + r)

        for r in range(NBUF):
            wait_gather(r, GPW - NBUF + r)
            start_out(r, GPW - NBUF + r)
        for r in range(NBUF):
            wait_out(r, GPW - NBUF + r)

    return gather_kernel


_gather = _make_gather()


def kernel(text_vec, w2v_table):
    idx = jnp.pad(text_vec.astype(jnp.int32),
                  ((0, 0), (0, LP - L))).reshape(NGRP, G)
    out = _gather(idx, w2v_table)
    return out.reshape(B, LP, 2 * EMB)[:, :L, :EMB]


# speedup vs baseline: 1.0015x; 1.0010x over previous
import jax
import jax.numpy as jnp
from jax import lax
from jax.experimental import pallas as pl
from jax.experimental.pallas import tpu as pltpu
from jax.experimental.pallas import tpu_sc as plsc
import functools

VOCAB = 1000000
EMB = 64
B = 16384
L = 50
LP = 56

NC = 2
NS = 16
NW = NC * NS

G = 128
NGRP = B * LP // G
GPW = NGRP // NW
NBUF = 8


def _make_gather():
    mesh = plsc.VectorSubcoreMesh(
        core_axis_name="c", subcore_axis_name="s",
        num_cores=NC, num_subcores=NS)

    @functools.partial(
        pl.kernel,
        out_type=jax.ShapeDtypeStruct((NW, GPW, G, 2 * EMB), jnp.float32),
        mesh=mesh,
        scratch_types=[
            pltpu.VMEM((GPW, G), jnp.int32),
            pltpu.VMEM((NBUF, G, EMB), jnp.float32),
            pltpu.SemaphoreType.DMA((NBUF,)),
            pltpu.SemaphoreType.DMA((NBUF,)),
        ],
        compiler_params=pltpu.CompilerParams(use_tc_tiling_on_sc=False),
    )
    def gather_kernel(idx_hbm, table_hbm, out_hbm, idx_v, rows_v, gsem, osem):
        cid = lax.axis_index("c")
        sid = lax.axis_index("s")
        wid = sid * NC + cid
        g0 = wid * GPW
        pltpu.sync_copy(idx_hbm.at[pl.ds(g0, GPW)], idx_v)

        def start_gather(r, g):
            pltpu.make_async_copy(
                table_hbm.at[idx_v.at[g]], rows_v.at[r], gsem.at[r]).start()

        def wait_gather(r, g):
            pltpu.make_async_copy(
                table_hbm.at[idx_v.at[g]], rows_v.at[r], gsem.at[r]).wait()

        def start_out(r, g):
            pltpu.make_async_copy(
                rows_v.at[r, pl.ds(0, G // 2)],
                out_hbm.at[wid, g, pl.ds(0, G // 2), pl.ds(0, EMB)],
                osem.at[r]).start()
            pltpu.make_async_copy(
                rows_v.at[r, pl.ds(G // 2, G // 2)],
                out_hbm.at[wid, g, pl.ds(G // 2, G // 2), pl.ds(0, EMB)],
                osem.at[r]).start()

        def wait_out(r, g):
            pltpu.make_async_copy(
                rows_v.at[r, pl.ds(0, G // 2)],
                out_hbm.at[wid, g, pl.ds(0, G // 2), pl.ds(0, EMB)],
                osem.at[r]).wait()
            pltpu.make_async_copy(
                rows_v.at[r, pl.ds(G // 2, G // 2)],
                out_hbm.at[wid, g, pl.ds(G // 2, G // 2), pl.ds(0, EMB)],
                osem.at[r]).wait()

        for r in range(NBUF):
            start_gather(r, r)

        @pl.loop(0, GPW - NBUF, step=NBUF)
        def _(j0):
            for r in range(NBUF):
                wait_gather(r, j0 + r)
                start_out(r, j0 + r)
            for r in range(NBUF):
                wait_out(r, j0 + r)
                start_gather(r, j0 + NBUF + r)

        for r in range(NBUF):
            wait_gather(r, GPW - NBUF + r)
            start_out(r, GPW - NBUF + r)
        for r in range(NBUF):
            wait_out(r, GPW - NBUF + r)

    return gather_kernel


_gather = _make_gather()


def kernel(text_vec, w2v_table):
    idx = jnp.pad(text_vec.astype(jnp.int32),
                  ((0, 0), (0, LP - L))).reshape(NGRP, G)
    out = _gather(idx, w2v_table)
    return out.reshape(B, LP, 2 * EMB)[:, :L, :EMB]


# speedup vs baseline: 3.2912x; 3.2861x over previous
import jax
import jax.numpy as jnp
from jax import lax
from jax.experimental import pallas as pl
from jax.experimental.pallas import tpu as pltpu
from jax.experimental.pallas import tpu_sc as plsc
import functools

VOCAB = 1000000
EMB = 64
B = 16384
L = 50
LP = 56

NC = 2
NS = 16
NW = NC * NS

G = 128
NGRP = B * LP // G
GPW = NGRP // NW
NBUF = 8


def _make_gather():
    mesh = plsc.VectorSubcoreMesh(
        core_axis_name="c", subcore_axis_name="s",
        num_cores=NC, num_subcores=NS)

    @functools.partial(
        pl.kernel,
        out_type=jax.ShapeDtypeStruct((NW, GPW, G, 2 * EMB), jnp.float32),
        mesh=mesh,
        scratch_types=[
            pltpu.VMEM((GPW, G), jnp.int32),
            pltpu.VMEM((NBUF, G, EMB), jnp.float32),
            pltpu.SemaphoreType.DMA((NBUF,)),
            pltpu.SemaphoreType.DMA((NBUF,)),
        ],
        compiler_params=pltpu.CompilerParams(use_tc_tiling_on_sc=False),
    )
    def gather_kernel(idx_hbm, table_hbm, out_hbm, idx_v, rows_v, gsem, osem):
        cid = lax.axis_index("c")
        sid = lax.axis_index("s")
        wid = sid * NC + cid
        g0 = wid * GPW
        pltpu.sync_copy(idx_hbm.at[pl.ds(g0, GPW)], idx_v)

        def start_gather(r, g):
            pltpu.make_async_copy(
                table_hbm.at[idx_v.at[g]], rows_v.at[r], gsem.at[r]).start()

        def wait_gather(r, g):
            pltpu.make_async_copy(
                table_hbm.at[idx_v.at[g]], rows_v.at[r], gsem.at[r]).wait()

        def start_out(r, g):
            pltpu.make_async_copy(
                rows_v.at[r, pl.ds(0, G // 2)],
                out_hbm.at[wid, g, pl.ds(0, G // 2), pl.ds(0, EMB)],
                osem.at[r]).start()
            pltpu.make_async_copy(
                rows_v.at[r, pl.ds(G // 2, G // 2)],
                out_hbm.at[wid, g, pl.ds(G // 2, G // 2), pl.ds(0, EMB)],
                osem.at[r]).start()

        def wait_out(r, g):
            pltpu.make_async_copy(
                rows_v.at[r, pl.ds(0, G // 2)],
                out_hbm.at[wid, g, pl.ds(0, G // 2), pl.ds(0, EMB)],
                osem.at[r]).wait()
            pltpu.make_async_copy(
                rows_v.at[r, pl.ds(G // 2, G // 2)],
                out_hbm.at[wid, g, pl.ds(G // 2, G // 2), pl.ds(0, EMB)],
                osem.at[r]).wait()

        for r in range(NBUF):
            start_gather(r, r)

        @pl.loop(0, GPW - NBUF, step=NBUF)
        def _(j0):
            for r in range(NBUF):
                wait_gather(r, j0 + r)
                start_out(r, j0 + r)
            for r in range(NBUF):
                wait_out(r, j0 + r)
                start_gather(r, j0 + NBUF + r)

        for r in range(NBUF):
            wait_gather(r, GPW - NBUF + r)
            start_out(r, GPW - NBUF + r)
        for r in range(NBUF):
            wait_out(r, GPW - NBUF + r)

    return gather_kernel


_gather = _make_gather()


def kernel(text_vec, w2v_table):
    ti = text_vec.astype(jnp.int32)
    idx = jnp.concatenate([ti, ti[:, :LP - L]], axis=1).reshape(NGRP, G)
    out = _gather(idx, w2v_table)
    return out.reshape(B, LP, 2 * EMB)[:, :L, :EMB]
